# Initial kernel scaffold; baseline (speedup 1.0000x reference)
import functools

import jax
import jax.numpy as jnp
from jax import lax
from jax.experimental import pallas as pl
from jax.experimental.pallas import tpu as pltpu
from jax.experimental.pallas import tpu_sc as plsc

N = 10000
E = 320000
D = 128

NC = 2
NS = 16
L = 16
NW = NC * NS
EPW = E // NW
B = 80
NB = EPW // B
ROWS_PER_TILE = N // NS
ZCHUNK = 125
DCHUNK = 1000


def _sc_body(x_hbm, src_hbm, dst_hbm, hpart, dpart,
             sidx, didx, srows, drows, ebuf, pbuf, zbuf, dzero,
             h_sh, d_sh, sem0, sem1):
  c = lax.axis_index("c")
  s = lax.axis_index("s")
  wid = s * NC + c

  zv = jnp.zeros((L,), jnp.float32)

  def zrow(i, _):
    for j in range(D // L):
      zbuf[i, pl.ds(j * L, L)] = zv
    return 0
  lax.fori_loop(0, ZCHUNK, zrow, 0)

  def zd(i, _):
    dzero[pl.ds(i * L, L)] = zv
    return 0
  lax.fori_loop(0, DCHUNK // L, zd, 0)
  dzero[pl.ds(DCHUNK - L, L)] = zv

  for i in range(ROWS_PER_TILE // ZCHUNK):
    pltpu.sync_copy(zbuf, h_sh.at[pl.ds(s * ROWS_PER_TILE + i * ZCHUNK, ZCHUNK)])

  @pl.when(s < N // DCHUNK)
  def _():
    pltpu.sync_copy(dzero, d_sh.at[pl.ds(s * DCHUNK, DCHUNK)])

  plsc.subcore_barrier()

  def batch(b, _):
    base = wid * EPW + b * B
    pltpu.sync_copy(src_hbm.at[pl.ds(base, B)], sidx)
    pltpu.sync_copy(dst_hbm.at[pl.ds(base, B)], didx)
    c0 = pltpu.async_copy(x_hbm.at[sidx], srows, sem0)
    c1 = pltpu.async_copy(x_hbm.at[didx], drows, sem1)
    c0.wait()
    c1.wait()

    def dot(k, _):
      acc = srows[k, pl.ds(0, L)] * drows[k, pl.ds(0, L)]
      for j in range(1, D // L):
        acc = acc + srows[k, pl.ds(j * L, L)] * drows[k, pl.ds(j * L, L)]
      e = jnp.sum(acc)
      ebuf[k] = jnp.minimum(jnp.maximum(e, 0.0), 80.0)
      return 0
    lax.fori_loop(0, B, dot, 0)

    for g in range(B // L):
      pbuf[pl.ds(g * L, L)] = jnp.exp(ebuf[pl.ds(g * L, L)])

    def scale(k, _):
      pk = pbuf[k]
      for j in range(D // L):
        srows[k, pl.ds(j * L, L)] = srows[k, pl.ds(j * L, L)] * pk
      return 0
    lax.fori_loop(0, B, scale, 0)

    pltpu.sync_copy(srows, h_sh.at[didx], add=True)
    pltpu.sync_copy(pbuf, d_sh.at[didx], add=True)
    return 0

  lax.fori_loop(0, NB, batch, 0)

  plsc.subcore_barrier()

  for i in range(ROWS_PER_TILE // ZCHUNK):
    r0 = s * ROWS_PER_TILE + i * ZCHUNK
    pltpu.sync_copy(h_sh.at[pl.ds(r0, ZCHUNK)], hpart.at[c, pl.ds(r0, ZCHUNK)])

  @pl.when(s < N // DCHUNK)
  def _():
    pltpu.sync_copy(d_sh.at[pl.ds(s * DCHUNK, DCHUNK)],
                    dpart.at[c, pl.ds(s * DCHUNK, DCHUNK)])


def _combine_body(h_ref, d_ref, o_ref):
  hs = h_ref[0] + h_ref[1]
  d = d_ref[0] + d_ref[1]
  safe = jnp.where(d > 0, d, 1.0)
  o_ref[...] = jnp.maximum(jnp.where(d > 0, hs / safe, 0.0), 0.0)


@jax.jit
def kernel(x, edge_index):
  src = edge_index[0].astype(jnp.int32)
  dst = edge_index[1].astype(jnp.int32)

  mesh = plsc.VectorSubcoreMesh(core_axis_name="c", subcore_axis_name="s",
                                num_cores=NC, num_subcores=NS)
  sc_gat = pl.kernel(
      _sc_body,
      out_type=[jax.ShapeDtypeStruct((NC, N, D), jnp.float32),
                jax.ShapeDtypeStruct((NC, N), jnp.float32)],
      mesh=mesh,
      scratch_types=[
          pltpu.VMEM((B,), jnp.int32),
          pltpu.VMEM((B,), jnp.int32),
          pltpu.VMEM((B, D), jnp.float32),
          pltpu.VMEM((B, D), jnp.float32),
          pltpu.VMEM((B,), jnp.float32),
          pltpu.VMEM((B,), jnp.float32),
          pltpu.VMEM((ZCHUNK, D), jnp.float32),
          pltpu.VMEM((DCHUNK,), jnp.float32),
          pltpu.VMEM_SHARED((N, D), jnp.float32),
          pltpu.VMEM_SHARED((N,), jnp.float32),
          pltpu.SemaphoreType.DMA,
          pltpu.SemaphoreType.DMA,
      ],
  )
  hpart, dpart = sc_gat(x, src, dst)

  BN = 1000
  combine = pl.pallas_call(
      _combine_body,
      grid=(N // BN,),
      in_specs=[pl.BlockSpec((NC, BN, D), lambda i: (0, i, 0)),
                pl.BlockSpec((NC, BN, 1), lambda i: (0, i, 0))],
      out_specs=pl.BlockSpec((BN, D), lambda i: (i, 0)),
      out_shape=jax.ShapeDtypeStruct((N, D), jnp.float32),
  )
  return combine(hpart, dpart.reshape(NC, N, 1))


# speedup vs baseline: 2.3995x; 2.3995x over previous
import functools

import jax
import jax.numpy as jnp
from jax import lax
from jax.experimental import pallas as pl
from jax.experimental.pallas import tpu as pltpu
from jax.experimental.pallas import tpu_sc as plsc

N = 10000
E = 320000
D = 128

NC = 2
NS = 16
L = 16
NW = NC * NS
EPW = E // NW
B = 80
NB = EPW // B
CH = 200
NCH = N // CH
DCHUNK = 1000


def _sc_body(x_hbm, src_hbm, dst_hbm, hpart, dpart,
             sidx, didx, srows, drows, pbuf, zbuf, dzero,
             h_sh, d_sh, sem0, sem1):
  c = lax.axis_index("c")
  s = lax.axis_index("s")
  wid = s * NC + c

  zv = jnp.zeros((L,), jnp.float32)

  def zrow(i, _):
    for j in range(D // L):
      zbuf[i, pl.ds(j * L, L)] = zv
    return 0
  lax.fori_loop(0, CH, zrow, 0)

  def zd(i, _):
    dzero[pl.ds(i * L, L)] = zv
    return 0
  lax.fori_loop(0, DCHUNK // L, zd, 0)
  dzero[pl.ds(DCHUNK - L, L)] = zv

  for i in range((NCH + NS - 1) // NS):
    ch = s + i * NS
    @pl.when(ch < NCH)
    def _():
      pltpu.sync_copy(zbuf, h_sh.at[pl.ds(ch * CH, CH)])

  @pl.when(s < N // DCHUNK)
  def _():
    pltpu.sync_copy(dzero, d_sh.at[pl.ds(s * DCHUNK, DCHUNK)])

  plsc.subcore_barrier()

  def batch(b, _):
    base = wid * EPW + b * B
    pltpu.sync_copy(src_hbm.at[pl.ds(base, B)], sidx)
    pltpu.sync_copy(dst_hbm.at[pl.ds(base, B)], didx)
    c0 = pltpu.async_copy(x_hbm.at[sidx], srows, sem0)
    c1 = pltpu.async_copy(x_hbm.at[didx], drows, sem1)
    c0.wait()
    c1.wait()

    lanes = lax.iota(jnp.int32, L)
    one = jnp.full((L,), 1, jnp.int32)
    zi = jnp.zeros((L,), jnp.int32)
    zf = jnp.zeros((L,), jnp.float32)
    UF = 16
    UFS = 8
    for g in range(B // L):
      rvec = lanes + g * L
      def dot(i, carry):
        fv, a0, a1, a2, a3 = carry
        accs = [a0, a1, a2, a3]
        for u in range(UF):
          sv = plsc.load_gather(srows, [rvec, fv])
          dv = plsc.load_gather(drows, [rvec, fv])
          accs[u % 4] = accs[u % 4] + sv * dv
          fv = fv + one
        return (fv, accs[0], accs[1], accs[2], accs[3])
      _, a0, a1, a2, a3 = lax.fori_loop(0, D // UF, dot, (zi, zf, zf, zf, zf))
      ev = (a0 + a1) + (a2 + a3)
      ev = jnp.minimum(jnp.maximum(ev, 0.0), 80.0)
      pbuf[pl.ds(g * L, L)] = jnp.exp(ev)

    for g in range(B // L):
      pv = pbuf[pl.ds(g * L, L)]
      rvec = lanes + g * L
      def scale(i, fv):
        for u in range(UFS):
          sv = plsc.load_gather(srows, [rvec, fv])
          plsc.store_scatter(srows, [rvec, fv], sv * pv)
          fv = fv + one
        return fv
      lax.fori_loop(0, D // UFS, scale, zi)

    pltpu.sync_copy(srows, h_sh.at[didx], add=True)
    pltpu.sync_copy(pbuf, d_sh.at[didx], add=True)
    return 0

  lax.fori_loop(0, NB, batch, 0)

  plsc.subcore_barrier()

  for i in range((NCH + NS - 1) // NS):
    ch = s + i * NS
    @pl.when(ch < NCH)
    def _():
      pltpu.sync_copy(h_sh.at[pl.ds(ch * CH, CH)], hpart.at[c, pl.ds(ch * CH, CH)])

  @pl.when(s < N // DCHUNK)
  def _():
    pltpu.sync_copy(d_sh.at[pl.ds(s * DCHUNK, DCHUNK)], dzero)
    pltpu.sync_copy(dzero, dpart.at[pl.ds(c * N + s * DCHUNK, DCHUNK)])


def _combine_body(h_ref, d_ref, o_ref):
  hs = h_ref[0] + h_ref[1]
  d = d_ref[0] + d_ref[1]
  safe = jnp.where(d > 0, d, 1.0)
  o_ref[...] = jnp.maximum(jnp.where(d > 0, hs / safe, 0.0), 0.0)


@jax.jit
def kernel(x, edge_index):
  src = edge_index[0].astype(jnp.int32)
  dst = edge_index[1].astype(jnp.int32)

  mesh = plsc.VectorSubcoreMesh(core_axis_name="c", subcore_axis_name="s",
                                num_cores=NC, num_subcores=NS)
  sc_gat = pl.kernel(
      _sc_body,
      out_type=[jax.ShapeDtypeStruct((NC, N, D), jnp.float32),
                jax.ShapeDtypeStruct((NC * N,), jnp.float32)],
      mesh=mesh,
      scratch_types=[
          pltpu.VMEM((B,), jnp.int32),
          pltpu.VMEM((B,), jnp.int32),
          pltpu.VMEM((B, D), jnp.float32),
          pltpu.VMEM((B, D), jnp.float32),
          pltpu.VMEM((B,), jnp.float32),
          pltpu.VMEM((CH, D), jnp.float32),
          pltpu.VMEM((DCHUNK,), jnp.float32),
          pltpu.VMEM_SHARED((N, D), jnp.float32),
          pltpu.VMEM_SHARED((N,), jnp.float32),
          pltpu.SemaphoreType.DMA,
          pltpu.SemaphoreType.DMA,
      ],
  )
  hpart, dpart = sc_gat(x, src, dst)

  BN = 1000
  combine = pl.pallas_call(
      _combine_body,
      grid=(N // BN,),
      in_specs=[pl.BlockSpec((NC, BN, D), lambda i: (0, i, 0)),
                pl.BlockSpec((NC, BN, 1), lambda i: (0, i, 0))],
      out_specs=pl.BlockSpec((BN, D), lambda i: (i, 0)),
      out_shape=jax.ShapeDtypeStruct((N, D), jnp.float32),
  )
  return combine(hpart, dpart.reshape(NC, N, 1))


# speedup vs baseline: 4.3882x; 1.8288x over previous
import functools

import jax
import jax.numpy as jnp
from jax import lax
from jax.experimental import pallas as pl
from jax.experimental.pallas import tpu as pltpu
from jax.experimental.pallas import tpu_sc as plsc

N = 10000
E = 320000
D = 128

NC = 2
NS = 16
L = 16
NW = NC * NS
EPW = E // NW
B = 80
NB = EPW // B
CH = 200
NCH = N // CH
DCHUNK = 1000


def _sc_body(x_hbm, src_hbm, dst_hbm, hpart, dpart,
             sidx, didx, srows, drows, pbuf, zbuf, dzero,
             h_sh, d_sh, sem0, sem1):
  c = lax.axis_index("c")
  s = lax.axis_index("s")
  wid = s * NC + c

  zv = jnp.zeros((L,), jnp.float32)

  def zrow(i, _):
    for j in range(D // L):
      zbuf[i, pl.ds(j * L, L)] = zv
    return 0
  lax.fori_loop(0, CH, zrow, 0)

  def zd(i, _):
    dzero[pl.ds(i * L, L)] = zv
    return 0
  lax.fori_loop(0, DCHUNK // L, zd, 0)
  dzero[pl.ds(DCHUNK - L, L)] = zv

  for i in range((NCH + NS - 1) // NS):
    ch = s + i * NS
    @pl.when(ch < NCH)
    def _():
      pltpu.sync_copy(zbuf, h_sh.at[pl.ds(ch * CH, CH)])

  @pl.when(s < N // DCHUNK)
  def _():
    pltpu.sync_copy(dzero, d_sh.at[pl.ds(s * DCHUNK, DCHUNK)])

  plsc.subcore_barrier()

  def batch(b, _):
    base = wid * EPW + b * B
    pltpu.sync_copy(src_hbm.at[pl.ds(base, B)], sidx)
    pltpu.sync_copy(dst_hbm.at[pl.ds(base, B)], didx)
    c0 = pltpu.async_copy(x_hbm.at[sidx], srows, sem0)
    c1 = pltpu.async_copy(x_hbm.at[didx], drows, sem1)
    c0.wait()
    c1.wait()

    lanes = lax.iota(jnp.int32, L)
    one = jnp.full((L,), 1, jnp.int32)
    zi = jnp.zeros((L,), jnp.int32)
    zf = jnp.zeros((L,), jnp.float32)
    UF = 16
    UFS = 8
    for g in range(B // L):
      rvec = lanes + g * L
      def dot(i, carry):
        fv, a0, a1, a2, a3 = carry
        accs = [a0, a1, a2, a3]
        for u in range(UF):
          sv = plsc.load_gather(srows, [rvec, fv])
          dv = plsc.load_gather(drows, [rvec, fv])
          accs[u % 4] = accs[u % 4] + sv * dv
          fv = fv + one
        return (fv, accs[0], accs[1], accs[2], accs[3])
      _, a0, a1, a2, a3 = lax.fori_loop(0, D // UF, dot, (zi, zf, zf, zf, zf))
      ev = (a0 + a1) + (a2 + a3)
      ev = jnp.minimum(jnp.maximum(ev, 0.0), 80.0)
      pbuf[pl.ds(g * L, L)] = jnp.exp(ev)

    def scale(k, _):
      pkv = plsc.load_gather(pbuf, [jnp.full((L,), k, jnp.int32)])
      for j in range(D // L):
        srows[k, pl.ds(j * L, L)] = srows[k, pl.ds(j * L, L)] * pkv
      return 0
    lax.fori_loop(0, B, scale, 0)

    pltpu.sync_copy(srows, h_sh.at[didx], add=True)
    pltpu.sync_copy(pbuf, d_sh.at[didx], add=True)
    return 0

  lax.fori_loop(0, NB, batch, 0)

  plsc.subcore_barrier()

  for i in range((NCH + NS - 1) // NS):
    ch = s + i * NS
    @pl.when(ch < NCH)
    def _():
      pltpu.sync_copy(h_sh.at[pl.ds(ch * CH, CH)], hpart.at[c, pl.ds(ch * CH, CH)])

  @pl.when(s < N // DCHUNK)
  def _():
    pltpu.sync_copy(d_sh.at[pl.ds(s * DCHUNK, DCHUNK)], dzero)
    pltpu.sync_copy(dzero, dpart.at[pl.ds(c * N + s * DCHUNK, DCHUNK)])


def _combine_body(h_ref, d_ref, o_ref):
  hs = h_ref[0] + h_ref[1]
  d = d_ref[0] + d_ref[1]
  safe = jnp.where(d > 0, d, 1.0)
  o_ref[...] = jnp.maximum(jnp.where(d > 0, hs / safe, 0.0), 0.0)


@jax.jit
def kernel(x, edge_index):
  src = edge_index[0].astype(jnp.int32)
  dst = edge_index[1].astype(jnp.int32)

  mesh = plsc.VectorSubcoreMesh(core_axis_name="c", subcore_axis_name="s",
                                num_cores=NC, num_subcores=NS)
  sc_gat = pl.kernel(
      _sc_body,
      out_type=[jax.ShapeDtypeStruct((NC, N, D), jnp.float32),
                jax.ShapeDtypeStruct((NC * N,), jnp.float32)],
      mesh=mesh,
      scratch_types=[
          pltpu.VMEM((B,), jnp.int32),
          pltpu.VMEM((B,), jnp.int32),
          pltpu.VMEM((B, D), jnp.float32),
          pltpu.VMEM((B, D), jnp.float32),
          pltpu.VMEM((B,), jnp.float32),
          pltpu.VMEM((CH, D), jnp.float32),
          pltpu.VMEM((DCHUNK,), jnp.float32),
          pltpu.VMEM_SHARED((N, D), jnp.float32),
          pltpu.VMEM_SHARED((N,), jnp.float32),
          pltpu.SemaphoreType.DMA,
          pltpu.SemaphoreType.DMA,
      ],
  )
  hpart, dpart = sc_gat(x, src, dst)

  BN = 1000
  combine = pl.pallas_call(
      _combine_body,
      grid=(N // BN,),
      in_specs=[pl.BlockSpec((NC, BN, D), lambda i: (0, i, 0)),
                pl.BlockSpec((NC, BN, 1), lambda i: (0, i, 0))],
      out_specs=pl.BlockSpec((BN, D), lambda i: (i, 0)),
      out_shape=jax.ShapeDtypeStruct((N, D), jnp.float32),
  )
  return combine(hpart, dpart.reshape(NC, N, 1))


# speedup vs baseline: 22.4430x; 5.1143x over previous
import jax
import jax.numpy as jnp
from jax import lax
from jax.experimental import pallas as pl
from jax.experimental.pallas import tpu as pltpu
from jax.experimental.pallas import tpu_sc as plsc

N = 10000
E = 320000
D = 128

NC = 2
NS = 16
L = 16
NW = NC * NS
EPW = E // NW
B = 80
NB = EPW // B
CH = 80
NCH = N // CH
DCHUNK = 1000


def _sc_body(x_hbm, eidx_hbm, hpart, dpart,
             idx0, idx1, srows0, srows1, drows0, drows1, pbuf, dzero, didxbuf,
             h_sh, d_sh, semI0, semI1, semS0, semS1, semD0, semD1):
  c = lax.axis_index("c")
  s = lax.axis_index("s")
  wid = s * NC + c

  zv = jnp.zeros((L,), jnp.float32)

  def zrow(i, _):
    for j in range(D // L):
      srows0[i, pl.ds(j * L, L)] = zv
    return 0
  lax.fori_loop(0, B, zrow, 0)

  def zd(i, _):
    dzero[pl.ds(i * L, L)] = zv
    return 0
  lax.fori_loop(0, DCHUNK // L, zd, 0)
  dzero[pl.ds(DCHUNK - L, L)] = zv

  for i in range((NCH + NS - 1) // NS):
    ch = s + i * NS
    @pl.when(ch < NCH)
    def _():
      pltpu.sync_copy(srows0, h_sh.at[pl.ds(ch * CH, CH)])

  @pl.when(s < N // DCHUNK)
  def _():
    pltpu.sync_copy(dzero, d_sh.at[pl.ds(s * DCHUNK, DCHUNK)])

  plsc.subcore_barrier()

  idx_ = (idx0, idx1)
  srows_ = (srows0, srows1)
  drows_ = (drows0, drows1)
  semI = (semI0, semI1)
  semS = (semS0, semS1)
  semD = (semD0, semD1)

  def fire_idx(b, slot):
    pltpu.async_copy(eidx_hbm.at[wid, b], idx_[slot], semI[slot])

  def wait_idx(b, slot):
    pltpu.make_async_copy(eidx_hbm.at[wid, b], idx_[slot], semI[slot]).wait()

  def fire_rows(b, slot):
    pltpu.async_copy(x_hbm.at[idx_[slot].at[0]], srows_[slot], semS[slot])
    pltpu.async_copy(x_hbm.at[idx_[slot].at[1]], drows_[slot], semD[slot])

  def wait_rows(b, slot):
    pltpu.make_async_copy(x_hbm.at[idx_[slot].at[0]], srows_[slot], semS[slot]).wait()
    pltpu.make_async_copy(x_hbm.at[idx_[slot].at[1]], drows_[slot], semD[slot]).wait()

  def compute(b, slot):
    srows = srows_[slot]
    drows = drows_[slot]
    lanes = lax.iota(jnp.int32, L)
    zf = jnp.zeros((L,), jnp.float32)
    EU = 2
    for g in range(B // L):
      def dot(i, ev):
        for u in range(EU):
          k = i * EU + u
          row = g * L + k
          ms = [srows[row, pl.ds(j * L, L)] * drows[row, pl.ds(j * L, L)]
                for j in range(D // L)]
          while len(ms) > 1:
            ms = [ms[2 * t] + ms[2 * t + 1] for t in range(len(ms) // 2)]
          e = jnp.sum(ms[0])
          ev = jnp.where(lanes == k, e, ev)
        return ev
      ev = lax.fori_loop(0, L // EU, dot, zf)
      ev = jnp.minimum(jnp.maximum(ev, 0.0), 80.0)
      pbuf[pl.ds(g * L, L)] = jnp.exp(ev)

    def scale(k, _):
      pkv = plsc.load_gather(pbuf, [jnp.full((L,), k, jnp.int32)])
      for j in range(D // L):
        srows[k, pl.ds(j * L, L)] = srows[k, pl.ds(j * L, L)] * pkv
      return 0
    lax.fori_loop(0, B, scale, 0)

    pltpu.sync_copy(srows, h_sh.at[didxbuf], add=True)
    pltpu.sync_copy(pbuf, d_sh.at[didxbuf], add=True)

  pltpu.sync_copy(eidx_hbm.at[wid, 0], idx0)
  pltpu.sync_copy(eidx_hbm.at[wid, 1], idx1)
  fire_rows(0, 0)
  fire_rows(1, 1)

  def one(b, slot):
    @pl.when(b < NB)
    def _():
      wait_rows(b, slot)
      for g2 in range(B // L):
        didxbuf[pl.ds(g2 * L, L)] = idx_[slot][1, pl.ds(g2 * L, L)]
      @pl.when(b + 2 < NB)
      def _():
        fire_idx(b + 2, slot)
      compute(b, slot)
      @pl.when(b + 2 < NB)
      def _():
        wait_idx(b + 2, slot)
        fire_rows(b + 2, slot)

  def pair(t, _):
    b = 2 * t
    one(b, 0)
    one(b + 1, 1)
    return 0

  lax.fori_loop(0, (NB + 1) // 2, pair, 0)

  plsc.subcore_barrier()

  for i in range((NCH + NS - 1) // NS):
    ch = s + i * NS
    @pl.when(ch < NCH)
    def _():
      pltpu.sync_copy(h_sh.at[pl.ds(ch * CH, CH)], hpart.at[c, pl.ds(ch * CH, CH)])

  @pl.when(s < N // DCHUNK)
  def _():
    pltpu.sync_copy(d_sh.at[pl.ds(s * DCHUNK, DCHUNK)], dzero)
    pltpu.sync_copy(dzero, dpart.at[pl.ds(c * N + s * DCHUNK, DCHUNK)])


def _combine_body(h_ref, d_ref, o_ref):
  hs = h_ref[0] + h_ref[1]
  d = d_ref[0] + d_ref[1]
  safe = jnp.where(d > 0, d, 1.0)
  o_ref[...] = jnp.maximum(jnp.where(d > 0, hs / safe, 0.0), 0.0)


@jax.jit
def kernel(x, edge_index):
  eidx = (edge_index.astype(jnp.int32)
          .reshape(2, NW, NB, B).transpose(1, 2, 0, 3))

  mesh = plsc.VectorSubcoreMesh(core_axis_name="c", subcore_axis_name="s",
                                num_cores=NC, num_subcores=NS)
  sc_gat = pl.kernel(
      _sc_body,
      out_type=[jax.ShapeDtypeStruct((NC, N, D), jnp.float32),
                jax.ShapeDtypeStruct((NC * N,), jnp.float32)],
      mesh=mesh,
      compiler_params=pltpu.CompilerParams(needs_layout_passes=False),
      scratch_types=[
          pltpu.VMEM((2, B), jnp.int32),
          pltpu.VMEM((2, B), jnp.int32),
          pltpu.VMEM((B, D), jnp.float32),
          pltpu.VMEM((B, D), jnp.float32),
          pltpu.VMEM((B, D), jnp.float32),
          pltpu.VMEM((B, D), jnp.float32),
          pltpu.VMEM((B,), jnp.float32),
          pltpu.VMEM((DCHUNK,), jnp.float32),
          pltpu.VMEM((B,), jnp.int32),
          pltpu.VMEM_SHARED((N, D), jnp.float32),
          pltpu.VMEM_SHARED((N,), jnp.float32),
          pltpu.SemaphoreType.DMA,
          pltpu.SemaphoreType.DMA,
          pltpu.SemaphoreType.DMA,
          pltpu.SemaphoreType.DMA,
          pltpu.SemaphoreType.DMA,
          pltpu.SemaphoreType.DMA,
      ],
  )
  hpart, dpart = sc_gat(x, eidx)

  BN = 1000
  combine = pl.pallas_call(
      _combine_body,
      grid=(N // BN,),
      in_specs=[pl.BlockSpec((NC, BN, D), lambda i: (0, i, 0)),
                pl.BlockSpec((NC, BN, 1), lambda i: (0, i, 0))],
      out_specs=pl.BlockSpec((BN, D), lambda i: (i, 0)),
      out_shape=jax.ShapeDtypeStruct((N, D), jnp.float32),
  )
  return combine(hpart, dpart.reshape(NC, N, 1))
